# baseline (speedup 1.0000x reference)
import functools

import jax
import jax.numpy as jnp
import numpy as np
from jax import lax
from jax.experimental import pallas as pl
from jax.experimental.pallas import tpu as pltpu
from jax.experimental.pallas import tpu_sc as plsc

N = 10000
E = 160000
D = 128
H = 8
QK = 32
HD = 256

NC = 2
NS = 16
NW = NC * NS
NPAD = 10240
NP8 = 1280
NP2 = 5120

_f32 = jnp.float32
_bf16 = jnp.bfloat16
_i32 = jnp.int32


@functools.cache
def _mesh():
    return plsc.VectorSubcoreMesh(
        core_axis_name="c", subcore_axis_name="s",
        num_cores=NC, num_subcores=NS)


def _edge_body(e_ref, eqw, eqb, ekw, ekb, evw, evb,
               p1w, p1b, o1w, o1b, p2w, p2b, o2w, o2b, rz,
               eq_o, ek_o, ev_o, e2_o):
    e = e_ref[...]
    eb = e.astype(_bf16)
    eq_o[...] = jnp.dot(eb, eqw[...], preferred_element_type=_f32) + eqb[...]
    ek_o[...] = jnp.dot(eb, ekw[...], preferred_element_type=_f32) + ekb[...]
    ev_o[...] = jnp.dot(eb, evw[...], preferred_element_type=_f32) + evb[...]
    r = rz[0, 0]
    h1 = jnp.dot(eb, p1w[...], preferred_element_type=_f32) + p1b[...]
    a1 = h1[:, :HD] * jax.nn.relu(h1[:, HD:])
    f1 = jnp.dot(a1.astype(_bf16), o1w[...],
                 preferred_element_type=_f32) + o1b[...]
    e1 = e + r * f1
    h2 = jnp.dot(e1.astype(_bf16), p2w[...],
                 preferred_element_type=_f32) + p2b[...]
    a2 = h2[:, :HD] * jax.nn.relu(h2[:, HD:])
    f2 = jnp.dot(a2.astype(_bf16), o2w[...],
                 preferred_element_type=_f32) + o2b[...]
    e2_o[...] = e1 + r * f2


def _edge_tc(e, eqw, eqb, ekw, ekb, evw, evb,
             p1w, p1b, o1w, o1b, p2w, p2b, o2w, o2b, rz):
    R = 640
    row = pl.BlockSpec((R, D), lambda i: (i, 0))
    row_hd = pl.BlockSpec((R, HD), lambda i: (i, 0))

    def const(shape):
        return pl.BlockSpec(shape, lambda i: tuple(0 for _ in shape))

    return pl.pallas_call(
        _edge_body,
        grid=(E // R,),
        in_specs=[row,
                  const((D, HD)), const((1, HD)),
                  const((D, HD)), const((1, HD)),
                  const((D, HD)), const((1, HD)),
                  const((D, 4 * D)), const((1, 4 * D)),
                  const((HD, D)), const((1, D)),
                  const((D, 4 * D)), const((1, 4 * D)),
                  const((HD, D)), const((1, D)),
                  const((1, 1))],
        out_specs=[row_hd, row_hd, row_hd, row],
        out_shape=[jax.ShapeDtypeStruct((E, HD), _f32)] * 3
        + [jax.ShapeDtypeStruct((E, D), _f32)],
    )(e, eqw, eqb, ekw, ekb, evw, evb,
      p1w, p1b, o1w, o1b, p2w, p2b, o2w, o2b, rz)


def _nodeproj_body(n_ref, qw, qb, kw, kb, vw, vb, q_o, k_o, v_o):
    nb = n_ref[...].astype(_bf16)
    q_o[...] = jnp.dot(nb, qw[...], preferred_element_type=_f32) + qb[...]
    k_o[...] = jnp.dot(nb, kw[...], preferred_element_type=_f32) + kb[...]
    v_o[...] = jnp.dot(nb, vw[...], preferred_element_type=_f32) + vb[...]


def _nodeproj_tc(n, qw, qb, kw, kb, vw, vb):
    return pl.pallas_call(
        _nodeproj_body,
        out_shape=[jax.ShapeDtypeStruct((N, HD), _f32)] * 3,
    )(n, qw, qb, kw, kb, vw, vb)


def _sc_gather_body(q_hbm, k_hbm, v_hbm, src1d, dst1d,
                    qd_o, ks_o, vs_o, sidx, didx, rows, sem):
    c = lax.axis_index("c")
    s = lax.axis_index("s")
    wid = s * NC + c
    nch = E // 32

    def chunk(j, _):
        r = wid + NW * j

        @pl.when(r < nch)
        def _():
            pltpu.sync_copy(dst1d.at[pl.ds(r * 32, 32)], didx)
            pltpu.sync_copy(src1d.at[pl.ds(r * 32, 32)], sidx)
            pltpu.async_copy(q_hbm.at[didx], rows, sem).wait()
            pltpu.sync_copy(rows, qd_o.at[pl.ds(r * 32, 32)])
            pltpu.async_copy(k_hbm.at[sidx], rows, sem).wait()
            pltpu.sync_copy(rows, ks_o.at[pl.ds(r * 32, 32)])
            pltpu.async_copy(v_hbm.at[sidx], rows, sem).wait()
            pltpu.sync_copy(rows, vs_o.at[pl.ds(r * 32, 32)])
        return 0

    lax.fori_loop(0, (nch + NW - 1) // NW, chunk, 0)


def _sc_gather(q, k, v, src1d, dst1d):
    return pl.kernel(
        _sc_gather_body,
        out_type=[jax.ShapeDtypeStruct((E, HD), _f32)] * 3,
        mesh=_mesh(),
        scratch_types=[
            pltpu.VMEM((32,), _i32),
            pltpu.VMEM((32,), _i32),
            pltpu.VMEM((32, HD), _f32),
            pltpu.SemaphoreType.DMA,
        ],
    )(q, k, v, src1d, dst1d)


def _score_body(qd, eq, ks, ek, m8, rA_o, rB_o, s8_o):
    z = (qd[...] + eq[...]) * (ks[...] + ek[...])
    R = z.shape[0]
    sel = (lax.broadcasted_iota(_i32, (HD, H), 0) // QK
           == lax.broadcasted_iota(_i32, (HD, H), 1)).astype(_f32)
    s = jnp.dot(z, sel, preferred_element_type=_f32) * np.float32(
        1.0 / np.sqrt(QK))
    s8_o[...] = s
    pad7 = jnp.zeros((R, 7), _f32)
    pad8 = jnp.zeros((R, 8), _f32)
    slotA = jnp.concatenate([s, jnp.ones((R, 1), _f32), pad7], axis=1)
    slotB = jnp.concatenate([s * s, pad8], axis=1)
    mask = (lax.broadcasted_iota(_i32, (R, 128), 1) // 16 == m8[...])
    tA = jnp.concatenate([slotA] * 8, axis=1)
    tB = jnp.concatenate([slotB] * 8, axis=1)
    rA_o[...] = jnp.where(mask, tA, 0.0)
    rB_o[...] = jnp.where(mask, tB, 0.0)


def _score_tc(qd, eq, ks, ek, m8):
    R = 640
    row_hd = pl.BlockSpec((R, HD), lambda i: (i, 0))
    return pl.pallas_call(
        _score_body,
        grid=(E // R,),
        in_specs=[row_hd] * 4 + [pl.BlockSpec((R, 1), lambda i: (i, 0))],
        out_specs=[pl.BlockSpec((R, 128), lambda i: (i, 0)),
                   pl.BlockSpec((R, 128), lambda i: (i, 0)),
                   pl.BlockSpec((R, H), lambda i: (i, 0))],
        out_shape=[jax.ShapeDtypeStruct((E, 128), _f32),
                   jax.ShapeDtypeStruct((E, 128), _f32),
                   jax.ShapeDtypeStruct((E, H), _f32)],
    )(qd, eq, ks, ek, m8)


def _sc_stats_body(rA_hbm, rB_hbm, dq8, outA, outB, acc, zbuf, rowsb, idxb):
    c = lax.axis_index("c")
    s = lax.axis_index("s")
    nps = NP8 // NS

    def zrow(i, _):
        for jj in range(8):
            zbuf[i, jj * 16:(jj + 1) * 16] = jnp.zeros((16,), _f32)
        return 0

    lax.fori_loop(0, 8, zrow, 0)
    nch = (E // 32) // NC

    for rows_hbm, out_hbm in ((rA_hbm, outA), (rB_hbm, outB)):
        def zcp(kk, _):
            pltpu.sync_copy(zbuf, acc.at[pl.ds(s * nps + kk * 8, 8)])
            return 0

        lax.fori_loop(0, nps // 8, zcp, 0)
        plsc.subcore_barrier()

        def chunk(j, _):
            t = s + NS * j

            @pl.when(t < nch)
            def _():
                r = c * nch + t
                pltpu.sync_copy(dq8.at[pl.ds(r * 32, 32)], idxb)
                pltpu.sync_copy(rows_hbm.at[pl.ds(r * 32, 32)], rowsb)
                pltpu.sync_copy(rowsb, acc.at[idxb], add=True)
            return 0

        lax.fori_loop(0, (nch + NS - 1) // NS, chunk, 0)
        plsc.subcore_barrier()
        pltpu.sync_copy(acc.at[pl.ds(s * nps, nps)],
                        out_hbm.at[c, pl.ds(s * nps, nps)])
        plsc.subcore_barrier()


def _sc_stats(rA, rB, dq8):
    return pl.kernel(
        _sc_stats_body,
        out_type=[jax.ShapeDtypeStruct((NC, NP8, 128), _f32)] * 2,
        mesh=_mesh(),
        scratch_types=[
            pltpu.VMEM_SHARED((NP8, 128), _f32),
            pltpu.VMEM((8, 128), _f32),
            pltpu.VMEM((32, 128), _f32),
            pltpu.VMEM((32,), _i32),
        ],
    )(rA, rB, dq8)


def _statsn_body(a_ref, b_ref, out):
    a = a_ref[0] + a_ref[1]
    b = b_ref[0] + b_ref[1]
    cnt = jnp.clip(a[:, 8:9], 1.0, None)
    mean = a[:, 0:8] / cnt
    msq = b[:, 0:8] / cnt
    var = jnp.clip(msq - mean * mean, 0.0, None)
    invstd = 1.0 / jnp.clip(jnp.sqrt(var), 1e-5, None)
    m16 = jnp.concatenate([mean, invstd], axis=1)
    place = (lax.broadcasted_iota(_i32, (16, D), 1)
             == lax.broadcasted_iota(_i32, (16, D), 0)).astype(_f32)
    out[...] = jnp.dot(m16, place, preferred_element_type=_f32)


def _statsn_tc(a2, b2):
    Rn = 2048
    return pl.pallas_call(
        _statsn_body,
        grid=(NPAD // Rn,),
        in_specs=[pl.BlockSpec((2, Rn, 16), lambda i: (0, i, 0)),
                  pl.BlockSpec((2, Rn, 16), lambda i: (0, i, 0))],
        out_specs=pl.BlockSpec((Rn, D), lambda i: (i, 0)),
        out_shape=jax.ShapeDtypeStruct((NPAD, D), _f32),
    )(a2, b2)


def _sc_sgather_body(sn_hbm, dst1d, sg_o, didx, rows, sem):
    c = lax.axis_index("c")
    s = lax.axis_index("s")
    wid = s * NC + c
    nch = E // 32

    def chunk(j, _):
        r = wid + NW * j

        @pl.when(r < nch)
        def _():
            pltpu.sync_copy(dst1d.at[pl.ds(r * 32, 32)], didx)
            pltpu.async_copy(sn_hbm.at[didx], rows, sem).wait()
            pltpu.sync_copy(rows, sg_o.at[pl.ds(r * 32, 32)])
        return 0

    lax.fori_loop(0, (nch + NW - 1) // NW, chunk, 0)


def _sc_sgather(sn, dst1d):
    return pl.kernel(
        _sc_sgather_body,
        out_type=jax.ShapeDtypeStruct((E, D), _f32),
        mesh=_mesh(),
        scratch_types=[
            pltpu.VMEM((32,), _i32),
            pltpu.VMEM((32, D), _f32),
            pltpu.SemaphoreType.DMA,
        ],
    )(sn, dst1d)


def _attnw_body(s8, sg, vs, ev, ga, bi, m2, out):
    s = s8[...]
    mean = sg[:, 0:8]
    invstd = sg[:, 8:16]
    attn = ga[...] * (s - mean) * invstd + bi[...]
    rep = (lax.broadcasted_iota(_i32, (H, HD), 1) // QK
           == lax.broadcasted_iota(_i32, (H, HD), 0)).astype(_f32)
    w = jnp.dot(attn, rep, preferred_element_type=_f32) * (vs[...] + ev[...])
    mask = (lax.broadcasted_iota(_i32, (w.shape[0], 128), 1) // 64 == m2[...])
    for q in range(4):
        sl = w[:, q * 64:(q + 1) * 64]
        t2 = jnp.concatenate([sl, sl], axis=1)
        out[q] = jnp.where(mask, t2, 0.0)


def _attnw_tc(s8, sg, vs, ev, ga, bi, m2):
    R = 640
    return pl.pallas_call(
        _attnw_body,
        grid=(E // R,),
        in_specs=[pl.BlockSpec((R, H), lambda i: (i, 0)),
                  pl.BlockSpec((R, D), lambda i: (i, 0)),
                  pl.BlockSpec((R, HD), lambda i: (i, 0)),
                  pl.BlockSpec((R, HD), lambda i: (i, 0)),
                  pl.BlockSpec((1, H), lambda i: (0, 0)),
                  pl.BlockSpec((1, H), lambda i: (0, 0)),
                  pl.BlockSpec((R, 1), lambda i: (i, 0))],
        out_specs=pl.BlockSpec((4, R, 128), lambda i: (0, i, 0)),
        out_shape=jax.ShapeDtypeStruct((4, E, 128), _f32),
    )(s8, sg, vs, ev, ga, bi, m2)


def _sc_agg_body(w4_hbm, dq2, out_hbm, acc, zbuf, rowsb, idxb):
    c = lax.axis_index("c")
    s = lax.axis_index("s")
    nps = NP2 // NS

    def zrow(i, _):
        for jj in range(8):
            zbuf[i, jj * 16:(jj + 1) * 16] = jnp.zeros((16,), _f32)
        return 0

    lax.fori_loop(0, 8, zrow, 0)
    nch = E // 64

    for p in range(2):
        q = c + 2 * p

        def zcp(kk, _):
            pltpu.sync_copy(zbuf, acc.at[pl.ds(s * nps + kk * 8, 8)])
            return 0

        lax.fori_loop(0, nps // 8, zcp, 0)
        plsc.subcore_barrier()

        def chunk(j, _):
            r = s + NS * j

            @pl.when(r < nch)
            def _():
                pltpu.sync_copy(dq2.at[pl.ds(r * 64, 64)], idxb)
                pltpu.sync_copy(w4_hbm.at[q, pl.ds(r * 64, 64)], rowsb)
                pltpu.sync_copy(rowsb, acc.at[idxb], add=True)
            return 0

        lax.fori_loop(0, (nch + NS - 1) // NS, chunk, 0)
        plsc.subcore_barrier()
        pltpu.sync_copy(acc.at[pl.ds(s * nps, nps)],
                        out_hbm.at[q, pl.ds(s * nps, nps)])
        plsc.subcore_barrier()


def _sc_agg(w4, dq2):
    return pl.kernel(
        _sc_agg_body,
        out_type=jax.ShapeDtypeStruct((4, NP2, 128), _f32),
        mesh=_mesh(),
        scratch_types=[
            pltpu.VMEM_SHARED((NP2, 128), _f32),
            pltpu.VMEM((8, 128), _f32),
            pltpu.VMEM((64, 128), _f32),
            pltpu.VMEM((64,), _i32),
        ],
    )(w4, dq2)


def _aggstat_body(agg_ref, out):
    i = pl.program_id(0)
    a = agg_ref[...]
    s1 = jnp.sum(a, axis=0, keepdims=True)
    s2 = jnp.sum(a * a, axis=0, keepdims=True)
    upd = jnp.concatenate([s1, s2], axis=0)

    @pl.when(i == 0)
    def _():
        out[...] = jnp.zeros_like(out)

    out[...] += upd


def _aggstat_tc(agg2):
    Rn = 1000
    return pl.pallas_call(
        _aggstat_body,
        grid=(N // Rn,),
        in_specs=[pl.BlockSpec((Rn, HD), lambda i: (i, 0))],
        out_specs=pl.BlockSpec((2, HD), lambda i: (0, 0)),
        out_shape=jax.ShapeDtypeStruct((2, HD), _f32),
    )(agg2)


def _node_body(n_ref, agg_ref, stat, gnw, gnb, mixw, mixb,
               p1w, p1b, o1w, o1b, p2w, p2b, o2w, o2b, rz, out):
    agg = agg_ref[...]
    mean = stat[0:1] * np.float32(1.0 / N)
    msq = stat[1:2] * np.float32(1.0 / N)
    var = jnp.clip(msq - mean * mean, 0.0, None)
    std = jnp.sqrt(var + 1e-6)
    gn = gnw[...] * (agg - mean) / std + gnb[...]
    mixed = jnp.dot(jax.nn.relu(gn).astype(_bf16), mixw[...],
                    preferred_element_type=_f32) + mixb[...]
    r = rz[0, 0]
    n1 = n_ref[...] + r * mixed
    h1 = jnp.dot(n1.astype(_bf16), p1w[...],
                 preferred_element_type=_f32) + p1b[...]
    a1 = h1[:, :HD] * jax.nn.relu(h1[:, HD:])
    f1 = jnp.dot(a1.astype(_bf16), o1w[...],
                 preferred_element_type=_f32) + o1b[...]
    n2 = n1 + r * f1
    h2 = jnp.dot(n2.astype(_bf16), p2w[...],
                 preferred_element_type=_f32) + p2b[...]
    a2 = h2[:, :HD] * jax.nn.relu(h2[:, HD:])
    f2 = jnp.dot(a2.astype(_bf16), o2w[...],
                 preferred_element_type=_f32) + o2b[...]
    out[...] = n2 + r * f2


def _node_tc(n, agg2, stat, gnw, gnb, mixw, mixb,
             p1w, p1b, o1w, o1b, p2w, p2b, o2w, o2b, rz):
    Rn = 1000

    def const(shape):
        return pl.BlockSpec(shape, lambda i: tuple(0 for _ in shape))

    return pl.pallas_call(
        _node_body,
        grid=(N // Rn,),
        in_specs=[pl.BlockSpec((Rn, D), lambda i: (i, 0)),
                  pl.BlockSpec((Rn, HD), lambda i: (i, 0)),
                  const((2, HD)),
                  const((1, HD)), const((1, HD)),
                  const((HD, D)), const((1, D)),
                  const((D, 4 * D)), const((1, 4 * D)),
                  const((HD, D)), const((1, D)),
                  const((D, 4 * D)), const((1, 4 * D)),
                  const((HD, D)), const((1, D)),
                  const((1, 1))],
        out_specs=pl.BlockSpec((Rn, D), lambda i: (i, 0)),
        out_shape=jax.ShapeDtypeStruct((N, D), _f32),
    )(n, agg2, stat, gnw, gnb, mixw, mixb,
      p1w, p1b, o1w, o1b, p2w, p2b, o2w, o2b, rz)


def kernel(n, e, edge_index, q_w, q_b, k_w, k_b, v_w, v_b,
           eq_w, eq_b, ek_w, ek_b, ev_w, ev_b, gain, bias,
           node_ff_proj_w, node_ff_proj_b, node_ff_out_w, node_ff_out_b,
           edge_ff_proj_w, edge_ff_proj_b, edge_ff_out_w, edge_ff_out_b,
           node_ff2_proj_w, node_ff2_proj_b, node_ff2_out_w, node_ff2_out_b,
           edge_ff2_proj_w, edge_ff2_proj_b, edge_ff2_out_w, edge_ff2_out_b,
           gnw, gnb, mix_w, mix_b, rz_node, rz_edge):
    src1d = edge_index[0]
    dst1d = edge_index[1]
    m8 = (dst1d % 8).reshape(E, 1)
    m2 = (dst1d % 2).reshape(E, 1)
    dq8 = dst1d // 8
    dq2 = dst1d // 2
    row2 = lambda b: b.reshape(1, -1)
    rzn = rz_node.reshape(1, 1)
    rze = rz_edge.reshape(1, 1)

    bft = lambda w: w.T.astype(jnp.bfloat16)
    eq, ek, ev, e2 = _edge_tc(
        e, bft(eq_w), row2(eq_b), bft(ek_w), row2(ek_b), bft(ev_w),
        row2(ev_b),
        bft(edge_ff_proj_w), row2(edge_ff_proj_b),
        bft(edge_ff_out_w), row2(edge_ff_out_b),
        bft(edge_ff2_proj_w), row2(edge_ff2_proj_b),
        bft(edge_ff2_out_w), row2(edge_ff2_out_b), rze)

    q, k, v = _nodeproj_tc(n, bft(q_w), row2(q_b), bft(k_w), row2(k_b),
                           bft(v_w), row2(v_b))

    qd, ks, vs = _sc_gather(q, k, v, src1d, dst1d)

    rA, rB, s8 = _score_tc(qd, eq, ks, ek, m8)

    outA, outB = _sc_stats(rA, rB, dq8)
    a2 = outA.reshape(NC, NPAD, 16)
    b2 = outB.reshape(NC, NPAD, 16)

    sn = _statsn_tc(a2, b2)

    sg = _sc_sgather(sn, dst1d)

    w4 = _attnw_tc(s8, sg, vs, ev, gain.reshape(1, H), bias.reshape(1, H), m2)

    out4 = _sc_agg(w4, dq2)
    agg2 = (out4.reshape(4, NP2, 2, 64)
            .transpose(1, 2, 0, 3).reshape(NPAD, HD))

    stat = _aggstat_tc(agg2)

    n3 = _node_tc(n, agg2, stat, row2(gnw), row2(gnb), bft(mix_w),
                  row2(mix_b),
                  bft(node_ff_proj_w), row2(node_ff_proj_b),
                  bft(node_ff_out_w), row2(node_ff_out_b),
                  bft(node_ff2_proj_w), row2(node_ff2_proj_b),
                  bft(node_ff2_out_w), row2(node_ff2_out_b), rzn)

    return n3, e2


# speedup vs baseline: 1.0384x; 1.0384x over previous
import functools

import jax
import jax.numpy as jnp
import numpy as np
from jax import lax
from jax.experimental import pallas as pl
from jax.experimental.pallas import tpu as pltpu
from jax.experimental.pallas import tpu_sc as plsc

N = 10000
E = 160000
D = 128
H = 8
QK = 32
HD = 256

NC = 2
NS = 16
NW = NC * NS
NPAD = 10240
NP8 = 1280
NP2 = 5120

_f32 = jnp.float32
_i32 = jnp.int32


@functools.cache
def _mesh():
    return plsc.VectorSubcoreMesh(
        core_axis_name="c", subcore_axis_name="s",
        num_cores=NC, num_subcores=NS)


def _edge_body(e_ref, eqw, eqb, ekw, ekb, evw, evb,
               p1w, p1b, o1w, o1b, p2w, p2b, o2w, o2b, rz,
               eq_o, ek_o, ev_o, e2_o):
    e = e_ref[...]
    eq_o[...] = jnp.dot(e, eqw[...], preferred_element_type=_f32) + eqb[...]
    ek_o[...] = jnp.dot(e, ekw[...], preferred_element_type=_f32) + ekb[...]
    ev_o[...] = jnp.dot(e, evw[...], preferred_element_type=_f32) + evb[...]
    r = rz[0, 0]
    h1 = jnp.dot(e, p1w[...], preferred_element_type=_f32) + p1b[...]
    a1 = h1[:, :HD] * jax.nn.relu(h1[:, HD:])
    f1 = jnp.dot(a1, o1w[...], preferred_element_type=_f32) + o1b[...]
    e1 = e + r * f1
    h2 = jnp.dot(e1, p2w[...], preferred_element_type=_f32) + p2b[...]
    a2 = h2[:, :HD] * jax.nn.relu(h2[:, HD:])
    f2 = jnp.dot(a2, o2w[...], preferred_element_type=_f32) + o2b[...]
    e2_o[...] = e1 + r * f2


def _edge_tc(e, eqw, eqb, ekw, ekb, evw, evb,
             p1w, p1b, o1w, o1b, p2w, p2b, o2w, o2b, rz):
    R = 640
    row = pl.BlockSpec((R, D), lambda i: (i, 0))
    row_hd = pl.BlockSpec((R, HD), lambda i: (i, 0))

    def const(shape):
        return pl.BlockSpec(shape, lambda i: tuple(0 for _ in shape))

    return pl.pallas_call(
        _edge_body,
        grid=(E // R,),
        in_specs=[row,
                  const((D, HD)), const((1, HD)),
                  const((D, HD)), const((1, HD)),
                  const((D, HD)), const((1, HD)),
                  const((D, 4 * D)), const((1, 4 * D)),
                  const((HD, D)), const((1, D)),
                  const((D, 4 * D)), const((1, 4 * D)),
                  const((HD, D)), const((1, D)),
                  const((1, 1))],
        out_specs=[row_hd, row_hd, row_hd, row],
        out_shape=[jax.ShapeDtypeStruct((E, HD), _f32)] * 3
        + [jax.ShapeDtypeStruct((E, D), _f32)],
    )(e, eqw, eqb, ekw, ekb, evw, evb,
      p1w, p1b, o1w, o1b, p2w, p2b, o2w, o2b, rz)


def _nodeproj_body(n_ref, qw, qb, kw, kb, vw, vb, q_o, k_o, v_o):
    n = n_ref[...]
    q_o[...] = jnp.dot(n, qw[...], preferred_element_type=_f32) + qb[...]
    k_o[...] = jnp.dot(n, kw[...], preferred_element_type=_f32) + kb[...]
    v_o[...] = jnp.dot(n, vw[...], preferred_element_type=_f32) + vb[...]


def _nodeproj_tc(n, qw, qb, kw, kb, vw, vb):
    return pl.pallas_call(
        _nodeproj_body,
        out_shape=[jax.ShapeDtypeStruct((N, HD), _f32)] * 3,
    )(n, qw, qb, kw, kb, vw, vb)


def _sc_gather_body(q_hbm, k_hbm, v_hbm, src1d, dst1d,
                    qd_o, ks_o, vs_o, sidx, didx, rows, sem):
    c = lax.axis_index("c")
    s = lax.axis_index("s")
    wid = s * NC + c
    ngr = E // 128

    def group(j, _):
        g = wid + NW * j

        @pl.when(g < ngr)
        def _():
            pltpu.sync_copy(dst1d.at[pl.ds(g * 128, 128)], didx)
            pltpu.sync_copy(src1d.at[pl.ds(g * 128, 128)], sidx)
            for k in range(4):
                di = didx.at[pl.ds(k * 32, 32)]
                si = sidx.at[pl.ds(k * 32, 32)]
                base = g * 128 + k * 32
                pltpu.async_copy(q_hbm.at[di], rows, sem).wait()
                pltpu.sync_copy(rows, qd_o.at[pl.ds(base, 32)])
                pltpu.async_copy(k_hbm.at[si], rows, sem).wait()
                pltpu.sync_copy(rows, ks_o.at[pl.ds(base, 32)])
                pltpu.async_copy(v_hbm.at[si], rows, sem).wait()
                pltpu.sync_copy(rows, vs_o.at[pl.ds(base, 32)])
        return 0

    lax.fori_loop(0, (ngr + NW - 1) // NW, group, 0)


def _sc_gather(q, k, v, src1d, dst1d):
    return pl.kernel(
        _sc_gather_body,
        out_type=[jax.ShapeDtypeStruct((E, HD), _f32)] * 3,
        mesh=_mesh(),
        scratch_types=[
            pltpu.VMEM((128,), _i32),
            pltpu.VMEM((128,), _i32),
            pltpu.VMEM((32, HD), _f32),
            pltpu.SemaphoreType.DMA,
        ],
    )(q, k, v, src1d, dst1d)


def _score_body(qd, eq, ks, ek, m8, rA_o, rB_o, s8_o):
    z = (qd[...] + eq[...]) * (ks[...] + ek[...])
    R = z.shape[0]
    sel = (lax.broadcasted_iota(_i32, (HD, H), 0) // QK
           == lax.broadcasted_iota(_i32, (HD, H), 1)).astype(_f32)
    s = jnp.dot(z, sel, preferred_element_type=_f32) * np.float32(
        1.0 / np.sqrt(QK))
    s8_o[...] = s
    pad7 = jnp.zeros((R, 7), _f32)
    pad8 = jnp.zeros((R, 8), _f32)
    slotA = jnp.concatenate([s, jnp.ones((R, 1), _f32), pad7], axis=1)
    slotB = jnp.concatenate([s * s, pad8], axis=1)
    mask = (lax.broadcasted_iota(_i32, (R, 128), 1) // 16 == m8[...])
    tA = jnp.concatenate([slotA] * 8, axis=1)
    tB = jnp.concatenate([slotB] * 8, axis=1)
    rA_o[...] = jnp.where(mask, tA, 0.0)
    rB_o[...] = jnp.where(mask, tB, 0.0)


def _score_tc(qd, eq, ks, ek, m8):
    R = 640
    row_hd = pl.BlockSpec((R, HD), lambda i: (i, 0))
    return pl.pallas_call(
        _score_body,
        grid=(E // R,),
        in_specs=[row_hd] * 4 + [pl.BlockSpec((R, 1), lambda i: (i, 0))],
        out_specs=[pl.BlockSpec((R, 128), lambda i: (i, 0)),
                   pl.BlockSpec((R, 128), lambda i: (i, 0)),
                   pl.BlockSpec((R, H), lambda i: (i, 0))],
        out_shape=[jax.ShapeDtypeStruct((E, 128), _f32),
                   jax.ShapeDtypeStruct((E, 128), _f32),
                   jax.ShapeDtypeStruct((E, H), _f32)],
    )(qd, eq, ks, ek, m8)


def _sc_stats_body(rA_hbm, rB_hbm, dq8, outA, outB, acc, zbuf, rowsb, idxb):
    c = lax.axis_index("c")
    s = lax.axis_index("s")
    nps = NP8 // NS

    def zrow(i, _):
        for jj in range(8):
            zbuf[i, jj * 16:(jj + 1) * 16] = jnp.zeros((16,), _f32)
        return 0

    lax.fori_loop(0, 8, zrow, 0)
    nch = (E // 32) // NC

    for rows_hbm, out_hbm in ((rA_hbm, outA), (rB_hbm, outB)):
        def zcp(kk, _):
            pltpu.sync_copy(zbuf, acc.at[pl.ds(s * nps + kk * 8, 8)])
            return 0

        lax.fori_loop(0, nps // 8, zcp, 0)
        plsc.subcore_barrier()

        def chunk(j, _):
            t = s + NS * j

            @pl.when(t < nch)
            def _():
                r = c * nch + t
                pltpu.sync_copy(dq8.at[pl.ds(r * 32, 32)], idxb)
                pltpu.sync_copy(rows_hbm.at[pl.ds(r * 32, 32)], rowsb)
                pltpu.sync_copy(rowsb, acc.at[idxb], add=True)
            return 0

        lax.fori_loop(0, (nch + NS - 1) // NS, chunk, 0)
        plsc.subcore_barrier()
        pltpu.sync_copy(acc.at[pl.ds(s * nps, nps)],
                        out_hbm.at[c, pl.ds(s * nps, nps)])
        plsc.subcore_barrier()


def _sc_stats(rA, rB, dq8):
    return pl.kernel(
        _sc_stats_body,
        out_type=[jax.ShapeDtypeStruct((NC, NP8, 128), _f32)] * 2,
        mesh=_mesh(),
        scratch_types=[
            pltpu.VMEM_SHARED((NP8, 128), _f32),
            pltpu.VMEM((8, 128), _f32),
            pltpu.VMEM((32, 128), _f32),
            pltpu.VMEM((32,), _i32),
        ],
    )(rA, rB, dq8)


def _statsn_body(a_ref, b_ref, out):
    a = a_ref[0] + a_ref[1]
    b = b_ref[0] + b_ref[1]
    cnt = jnp.clip(a[:, 8:9], 1.0, None)
    mean = a[:, 0:8] / cnt
    msq = b[:, 0:8] / cnt
    var = jnp.clip(msq - mean * mean, 0.0, None)
    invstd = 1.0 / jnp.clip(jnp.sqrt(var), 1e-5, None)
    m16 = jnp.concatenate([mean, invstd], axis=1)
    place = (lax.broadcasted_iota(_i32, (16, D), 1)
             == lax.broadcasted_iota(_i32, (16, D), 0)).astype(_f32)
    out[...] = jnp.dot(m16, place, preferred_element_type=_f32)


def _statsn_tc(a2, b2):
    Rn = 2048
    return pl.pallas_call(
        _statsn_body,
        grid=(NPAD // Rn,),
        in_specs=[pl.BlockSpec((2, Rn, 16), lambda i: (0, i, 0)),
                  pl.BlockSpec((2, Rn, 16), lambda i: (0, i, 0))],
        out_specs=pl.BlockSpec((Rn, D), lambda i: (i, 0)),
        out_shape=jax.ShapeDtypeStruct((NPAD, D), _f32),
    )(a2, b2)


def _sc_sgather_body(sn_hbm, dst1d, sg_o, didx, rows, sem):
    c = lax.axis_index("c")
    s = lax.axis_index("s")
    wid = s * NC + c
    ngr = E // 128

    def group(j, _):
        g = wid + NW * j

        @pl.when(g < ngr)
        def _():
            pltpu.sync_copy(dst1d.at[pl.ds(g * 128, 128)], didx)
            for k in range(4):
                di = didx.at[pl.ds(k * 32, 32)]
                base = g * 128 + k * 32
                pltpu.async_copy(sn_hbm.at[di], rows, sem).wait()
                pltpu.sync_copy(rows, sg_o.at[pl.ds(base, 32)])
        return 0

    lax.fori_loop(0, (ngr + NW - 1) // NW, group, 0)


def _sc_sgather(sn, dst1d):
    return pl.kernel(
        _sc_sgather_body,
        out_type=jax.ShapeDtypeStruct((E, D), _f32),
        mesh=_mesh(),
        scratch_types=[
            pltpu.VMEM((128,), _i32),
            pltpu.VMEM((32, D), _f32),
            pltpu.SemaphoreType.DMA,
        ],
    )(sn, dst1d)


def _attnw_body(s8, sg, vs, ev, ga, bi, m2, out):
    s = s8[...]
    mean = sg[:, 0:8]
    invstd = sg[:, 8:16]
    attn = ga[...] * (s - mean) * invstd + bi[...]
    rep = (lax.broadcasted_iota(_i32, (H, HD), 1) // QK
           == lax.broadcasted_iota(_i32, (H, HD), 0)).astype(_f32)
    w = jnp.dot(attn, rep, preferred_element_type=_f32) * (vs[...] + ev[...])
    mask = (lax.broadcasted_iota(_i32, (w.shape[0], 128), 1) // 64 == m2[...])
    for q in range(4):
        sl = w[:, q * 64:(q + 1) * 64]
        t2 = jnp.concatenate([sl, sl], axis=1)
        out[q] = jnp.where(mask, t2, 0.0)


def _attnw_tc(s8, sg, vs, ev, ga, bi, m2):
    R = 640
    return pl.pallas_call(
        _attnw_body,
        grid=(E // R,),
        in_specs=[pl.BlockSpec((R, H), lambda i: (i, 0)),
                  pl.BlockSpec((R, D), lambda i: (i, 0)),
                  pl.BlockSpec((R, HD), lambda i: (i, 0)),
                  pl.BlockSpec((R, HD), lambda i: (i, 0)),
                  pl.BlockSpec((1, H), lambda i: (0, 0)),
                  pl.BlockSpec((1, H), lambda i: (0, 0)),
                  pl.BlockSpec((R, 1), lambda i: (i, 0))],
        out_specs=pl.BlockSpec((4, R, 128), lambda i: (0, i, 0)),
        out_shape=jax.ShapeDtypeStruct((4, E, 128), _f32),
    )(s8, sg, vs, ev, ga, bi, m2)


def _sc_agg_body(w4_hbm, dq2, out_hbm, acc, zbuf, rowsb, idxb):
    c = lax.axis_index("c")
    s = lax.axis_index("s")
    nps = NP2 // NS

    def zrow(i, _):
        for jj in range(8):
            zbuf[i, jj * 16:(jj + 1) * 16] = jnp.zeros((16,), _f32)
        return 0

    lax.fori_loop(0, 8, zrow, 0)
    nch = E // 64

    for p in range(2):
        q = c + 2 * p

        def zcp(kk, _):
            pltpu.sync_copy(zbuf, acc.at[pl.ds(s * nps + kk * 8, 8)])
            return 0

        lax.fori_loop(0, nps // 8, zcp, 0)
        plsc.subcore_barrier()

        def chunk(j, _):
            r = s + NS * j

            @pl.when(r < nch)
            def _():
                pltpu.sync_copy(dq2.at[pl.ds(r * 64, 64)], idxb)
                pltpu.sync_copy(w4_hbm.at[q, pl.ds(r * 64, 64)], rowsb)
                pltpu.sync_copy(rowsb, acc.at[idxb], add=True)
            return 0

        lax.fori_loop(0, (nch + NS - 1) // NS, chunk, 0)
        plsc.subcore_barrier()
        pltpu.sync_copy(acc.at[pl.ds(s * nps, nps)],
                        out_hbm.at[q, pl.ds(s * nps, nps)])
        plsc.subcore_barrier()


def _sc_agg(w4, dq2):
    return pl.kernel(
        _sc_agg_body,
        out_type=jax.ShapeDtypeStruct((4, NP2, 128), _f32),
        mesh=_mesh(),
        scratch_types=[
            pltpu.VMEM_SHARED((NP2, 128), _f32),
            pltpu.VMEM((8, 128), _f32),
            pltpu.VMEM((64, 128), _f32),
            pltpu.VMEM((64,), _i32),
        ],
    )(w4, dq2)


def _aggstat_body(agg_ref, out):
    i = pl.program_id(0)
    a = agg_ref[...]
    s1 = jnp.sum(a, axis=0, keepdims=True)
    s2 = jnp.sum(a * a, axis=0, keepdims=True)
    upd = jnp.concatenate([s1, s2], axis=0)

    @pl.when(i == 0)
    def _():
        out[...] = jnp.zeros_like(out)

    out[...] += upd


def _aggstat_tc(agg2):
    Rn = 1000
    return pl.pallas_call(
        _aggstat_body,
        grid=(N // Rn,),
        in_specs=[pl.BlockSpec((Rn, HD), lambda i: (i, 0))],
        out_specs=pl.BlockSpec((2, HD), lambda i: (0, 0)),
        out_shape=jax.ShapeDtypeStruct((2, HD), _f32),
    )(agg2)


def _node_body(n_ref, agg_ref, stat, gnw, gnb, mixw, mixb,
               p1w, p1b, o1w, o1b, p2w, p2b, o2w, o2b, rz, out):
    agg = agg_ref[...]
    mean = stat[0:1] * np.float32(1.0 / N)
    msq = stat[1:2] * np.float32(1.0 / N)
    var = jnp.clip(msq - mean * mean, 0.0, None)
    std = jnp.sqrt(var + 1e-6)
    gn = gnw[...] * (agg - mean) / std + gnb[...]
    mixed = jnp.dot(jax.nn.relu(gn), mixw[...],
                    preferred_element_type=_f32) + mixb[...]
    r = rz[0, 0]
    n1 = n_ref[...] + r * mixed
    h1 = jnp.dot(n1, p1w[...], preferred_element_type=_f32) + p1b[...]
    a1 = h1[:, :HD] * jax.nn.relu(h1[:, HD:])
    f1 = jnp.dot(a1, o1w[...], preferred_element_type=_f32) + o1b[...]
    n2 = n1 + r * f1
    h2 = jnp.dot(n2, p2w[...], preferred_element_type=_f32) + p2b[...]
    a2 = h2[:, :HD] * jax.nn.relu(h2[:, HD:])
    f2 = jnp.dot(a2, o2w[...], preferred_element_type=_f32) + o2b[...]
    out[...] = n2 + r * f2


def _node_tc(n, agg2, stat, gnw, gnb, mixw, mixb,
             p1w, p1b, o1w, o1b, p2w, p2b, o2w, o2b, rz):
    Rn = 1000

    def const(shape):
        return pl.BlockSpec(shape, lambda i: tuple(0 for _ in shape))

    return pl.pallas_call(
        _node_body,
        grid=(N // Rn,),
        in_specs=[pl.BlockSpec((Rn, D), lambda i: (i, 0)),
                  pl.BlockSpec((Rn, HD), lambda i: (i, 0)),
                  const((2, HD)),
                  const((1, HD)), const((1, HD)),
                  const((HD, D)), const((1, D)),
                  const((D, 4 * D)), const((1, 4 * D)),
                  const((HD, D)), const((1, D)),
                  const((D, 4 * D)), const((1, 4 * D)),
                  const((HD, D)), const((1, D)),
                  const((1, 1))],
        out_specs=pl.BlockSpec((Rn, D), lambda i: (i, 0)),
        out_shape=jax.ShapeDtypeStruct((N, D), _f32),
    )(n, agg2, stat, gnw, gnb, mixw, mixb,
      p1w, p1b, o1w, o1b, p2w, p2b, o2w, o2b, rz)


def kernel(n, e, edge_index, q_w, q_b, k_w, k_b, v_w, v_b,
           eq_w, eq_b, ek_w, ek_b, ev_w, ev_b, gain, bias,
           node_ff_proj_w, node_ff_proj_b, node_ff_out_w, node_ff_out_b,
           edge_ff_proj_w, edge_ff_proj_b, edge_ff_out_w, edge_ff_out_b,
           node_ff2_proj_w, node_ff2_proj_b, node_ff2_out_w, node_ff2_out_b,
           edge_ff2_proj_w, edge_ff2_proj_b, edge_ff2_out_w, edge_ff2_out_b,
           gnw, gnb, mix_w, mix_b, rz_node, rz_edge):
    src1d = edge_index[0]
    dst1d = edge_index[1]
    m8 = (dst1d % 8).reshape(E, 1)
    m2 = (dst1d % 2).reshape(E, 1)
    dq8 = dst1d // 8
    dq2 = dst1d // 2
    row2 = lambda b: b.reshape(1, -1)
    rzn = rz_node.reshape(1, 1)
    rze = rz_edge.reshape(1, 1)

    eq, ek, ev, e2 = _edge_tc(
        e, eq_w.T, row2(eq_b), ek_w.T, row2(ek_b), ev_w.T, row2(ev_b),
        edge_ff_proj_w.T, row2(edge_ff_proj_b),
        edge_ff_out_w.T, row2(edge_ff_out_b),
        edge_ff2_proj_w.T, row2(edge_ff2_proj_b),
        edge_ff2_out_w.T, row2(edge_ff2_out_b), rze)

    q, k, v = _nodeproj_tc(n, q_w.T, row2(q_b), k_w.T, row2(k_b),
                           v_w.T, row2(v_b))

    qd, ks, vs = _sc_gather(q, k, v, src1d, dst1d)

    rA, rB, s8 = _score_tc(qd, eq, ks, ek, m8)

    outA, outB = _sc_stats(rA, rB, dq8)
    a2 = outA.reshape(NC, NPAD, 16)
    b2 = outB.reshape(NC, NPAD, 16)

    sn = _statsn_tc(a2, b2)

    sg = _sc_sgather(sn, dst1d)

    w4 = _attnw_tc(s8, sg, vs, ev, gain.reshape(1, H), bias.reshape(1, H), m2)

    out4 = _sc_agg(w4, dq2)
    agg2 = (out4.reshape(4, NP2, 2, 64)
            .transpose(1, 2, 0, 3).reshape(NPAD, HD))

    stat = _aggstat_tc(agg2)

    n3 = _node_tc(n, agg2, stat, row2(gnw), row2(gnb), mix_w.T, row2(mix_b),
                  node_ff_proj_w.T, row2(node_ff_proj_b),
                  node_ff_out_w.T, row2(node_ff_out_b),
                  node_ff2_proj_w.T, row2(node_ff2_proj_b),
                  node_ff2_out_w.T, row2(node_ff2_out_b), rzn)

    return n3, e2


# speedup vs baseline: 1.0781x; 1.0382x over previous
import functools

import jax
import jax.numpy as jnp
import numpy as np
from jax import lax
from jax.experimental import pallas as pl
from jax.experimental.pallas import tpu as pltpu
from jax.experimental.pallas import tpu_sc as plsc

N = 10000
E = 160000
D = 128
H = 8
QK = 32
HD = 256

NC = 2
NS = 16
NW = NC * NS
NPAD = 10240
NP8 = 1280
NP2 = 5120

_f32 = jnp.float32
_i32 = jnp.int32


@functools.cache
def _mesh():
    return plsc.VectorSubcoreMesh(
        core_axis_name="c", subcore_axis_name="s",
        num_cores=NC, num_subcores=NS)


def _edge_body(e_ref, eqw, eqb, ekw, ekb, evw, evb,
               p1w, p1b, o1w, o1b, p2w, p2b, o2w, o2b, rz,
               eq_o, ek_o, ev_o, e2_o):
    e = e_ref[...]
    eq_o[...] = jnp.dot(e, eqw[...], preferred_element_type=_f32) + eqb[...]
    ek_o[...] = jnp.dot(e, ekw[...], preferred_element_type=_f32) + ekb[...]
    ev_o[...] = jnp.dot(e, evw[...], preferred_element_type=_f32) + evb[...]
    r = rz[0, 0]
    h1 = jnp.dot(e, p1w[...], preferred_element_type=_f32) + p1b[...]
    a1 = h1[:, :HD] * jax.nn.relu(h1[:, HD:])
    f1 = jnp.dot(a1, o1w[...], preferred_element_type=_f32) + o1b[...]
    e1 = e + r * f1
    h2 = jnp.dot(e1, p2w[...], preferred_element_type=_f32) + p2b[...]
    a2 = h2[:, :HD] * jax.nn.relu(h2[:, HD:])
    f2 = jnp.dot(a2, o2w[...], preferred_element_type=_f32) + o2b[...]
    e2_o[...] = e1 + r * f2


def _edge_tc(e, eqw, eqb, ekw, ekb, evw, evb,
             p1w, p1b, o1w, o1b, p2w, p2b, o2w, o2b, rz):
    R = 640
    row = pl.BlockSpec((R, D), lambda i: (i, 0))
    row_hd = pl.BlockSpec((R, HD), lambda i: (i, 0))

    def const(shape):
        return pl.BlockSpec(shape, lambda i: tuple(0 for _ in shape))

    return pl.pallas_call(
        _edge_body,
        grid=(E // R,),
        in_specs=[row,
                  const((D, HD)), const((1, HD)),
                  const((D, HD)), const((1, HD)),
                  const((D, HD)), const((1, HD)),
                  const((D, 4 * D)), const((1, 4 * D)),
                  const((HD, D)), const((1, D)),
                  const((D, 4 * D)), const((1, 4 * D)),
                  const((HD, D)), const((1, D)),
                  const((1, 1))],
        out_specs=[row_hd, row_hd, row_hd, row],
        out_shape=[jax.ShapeDtypeStruct((E, HD), _f32)] * 3
        + [jax.ShapeDtypeStruct((E, D), _f32)],
    )(e, eqw, eqb, ekw, ekb, evw, evb,
      p1w, p1b, o1w, o1b, p2w, p2b, o2w, o2b, rz)


def _nodeproj_body(n_ref, qw, qb, kw, kb, vw, vb, q_o, k_o, v_o):
    n = n_ref[...]
    q_o[...] = jnp.dot(n, qw[...], preferred_element_type=_f32) + qb[...]
    k_o[...] = jnp.dot(n, kw[...], preferred_element_type=_f32) + kb[...]
    v_o[...] = jnp.dot(n, vw[...], preferred_element_type=_f32) + vb[...]


def _nodeproj_tc(n, qw, qb, kw, kb, vw, vb):
    return pl.pallas_call(
        _nodeproj_body,
        out_shape=[jax.ShapeDtypeStruct((N, HD), _f32)] * 3,
    )(n, qw, qb, kw, kb, vw, vb)


def _sc_gather_body(q_hbm, k_hbm, v_hbm, src1d, dst1d,
                    qd_o, ks_o, vs_o, sidx, didx, b0, b1, g0, g1, w0, w1):
    c = lax.axis_index("c")
    s = lax.axis_index("s")
    wid = s * NC + c
    ngr = E // 128
    bufs = (b0, b1)
    gsems = (g0, g1)
    wsems = (w0, w1)

    def group(j, _):
        g = wid + NW * j

        @pl.when(g < ngr)
        def _():
            pltpu.sync_copy(dst1d.at[pl.ds(g * 128, 128)], didx)
            pltpu.sync_copy(src1d.at[pl.ds(g * 128, 128)], sidx)
            steps = []
            for k in range(8):
                di = didx.at[pl.ds(k * 16, 16)]
                si = sidx.at[pl.ds(k * 16, 16)]
                base = g * 128 + k * 16
                steps.append((q_hbm, di, qd_o, base))
                steps.append((k_hbm, si, ks_o, base))
                steps.append((v_hbm, si, vs_o, base))
            gds = [None, None]
            wds = [None, None]
            prev = None
            for t, (tbl, idx, out, base) in enumerate(steps):
                pa = t % 2
                if wds[pa] is not None:
                    wds[pa].wait()
                gds[pa] = pltpu.async_copy(tbl.at[idx], bufs[pa], gsems[pa])
                if prev is not None:
                    ptbl, pidx, pout, pbase = prev
                    gds[1 - pa].wait()
                    wds[1 - pa] = pltpu.async_copy(
                        bufs[1 - pa], pout.at[pl.ds(pbase, 16)], wsems[1 - pa])
                prev = (tbl, idx, out, base)
            gds[23 % 2].wait()
            wds[23 % 2] = pltpu.async_copy(
                bufs[23 % 2], steps[23][2].at[pl.ds(steps[23][3], 16)],
                wsems[23 % 2])
            wds[0].wait()
            wds[1].wait()
        return 0

    lax.fori_loop(0, (ngr + NW - 1) // NW, group, 0)


def _sc_gather(q, k, v, src1d, dst1d):
    return pl.kernel(
        _sc_gather_body,
        out_type=[jax.ShapeDtypeStruct((E, HD), _f32)] * 3,
        mesh=_mesh(),
        scratch_types=[
            pltpu.VMEM((128,), _i32),
            pltpu.VMEM((128,), _i32),
            pltpu.VMEM((16, HD), _f32),
            pltpu.VMEM((16, HD), _f32),
            pltpu.SemaphoreType.DMA,
            pltpu.SemaphoreType.DMA,
            pltpu.SemaphoreType.DMA,
            pltpu.SemaphoreType.DMA,
        ],
    )(q, k, v, src1d, dst1d)


def _score_body(qd, eq, ks, ek, m8, rA_o, rB_o, s8_o):
    z = (qd[...] + eq[...]) * (ks[...] + ek[...])
    R = z.shape[0]
    sel = (lax.broadcasted_iota(_i32, (HD, H), 0) // QK
           == lax.broadcasted_iota(_i32, (HD, H), 1)).astype(_f32)
    s = jnp.dot(z, sel, preferred_element_type=_f32) * np.float32(
        1.0 / np.sqrt(QK))
    s8_o[...] = s
    pad7 = jnp.zeros((R, 7), _f32)
    pad8 = jnp.zeros((R, 8), _f32)
    slotA = jnp.concatenate([s, jnp.ones((R, 1), _f32), pad7], axis=1)
    slotB = jnp.concatenate([s * s, pad8], axis=1)
    mask = (lax.broadcasted_iota(_i32, (R, 128), 1) // 16 == m8[...])
    tA = jnp.concatenate([slotA] * 8, axis=1)
    tB = jnp.concatenate([slotB] * 8, axis=1)
    rA_o[...] = jnp.where(mask, tA, 0.0)
    rB_o[...] = jnp.where(mask, tB, 0.0)


def _score_tc(qd, eq, ks, ek, m8):
    R = 640
    row_hd = pl.BlockSpec((R, HD), lambda i: (i, 0))
    return pl.pallas_call(
        _score_body,
        grid=(E // R,),
        in_specs=[row_hd] * 4 + [pl.BlockSpec((R, 1), lambda i: (i, 0))],
        out_specs=[pl.BlockSpec((R, 128), lambda i: (i, 0)),
                   pl.BlockSpec((R, 128), lambda i: (i, 0)),
                   pl.BlockSpec((R, H), lambda i: (i, 0))],
        out_shape=[jax.ShapeDtypeStruct((E, 128), _f32),
                   jax.ShapeDtypeStruct((E, 128), _f32),
                   jax.ShapeDtypeStruct((E, H), _f32)],
    )(qd, eq, ks, ek, m8)


def _sc_stats_body(rA_hbm, rB_hbm, dq8, outA, outB, acc, zbuf, rowsb, idxb):
    c = lax.axis_index("c")
    s = lax.axis_index("s")
    nps = NP8 // NS

    def zrow(i, _):
        for jj in range(8):
            zbuf[i, jj * 16:(jj + 1) * 16] = jnp.zeros((16,), _f32)
        return 0

    lax.fori_loop(0, 8, zrow, 0)
    nch = (E // 32) // NC

    for rows_hbm, out_hbm in ((rA_hbm, outA), (rB_hbm, outB)):
        def zcp(kk, _):
            pltpu.sync_copy(zbuf, acc.at[pl.ds(s * nps + kk * 8, 8)])
            return 0

        lax.fori_loop(0, nps // 8, zcp, 0)
        plsc.subcore_barrier()

        def chunk(j, _):
            t = s + NS * j

            @pl.when(t < nch)
            def _():
                r = c * nch + t
                pltpu.sync_copy(dq8.at[pl.ds(r * 32, 32)], idxb)
                pltpu.sync_copy(rows_hbm.at[pl.ds(r * 32, 32)], rowsb)
                pltpu.sync_copy(rowsb, acc.at[idxb], add=True)
            return 0

        lax.fori_loop(0, (nch + NS - 1) // NS, chunk, 0)
        plsc.subcore_barrier()
        pltpu.sync_copy(acc.at[pl.ds(s * nps, nps)],
                        out_hbm.at[c, pl.ds(s * nps, nps)])
        plsc.subcore_barrier()


def _sc_stats(rA, rB, dq8):
    return pl.kernel(
        _sc_stats_body,
        out_type=[jax.ShapeDtypeStruct((NC, NP8, 128), _f32)] * 2,
        mesh=_mesh(),
        scratch_types=[
            pltpu.VMEM_SHARED((NP8, 128), _f32),
            pltpu.VMEM((8, 128), _f32),
            pltpu.VMEM((32, 128), _f32),
            pltpu.VMEM((32,), _i32),
        ],
    )(rA, rB, dq8)


def _statsn_body(a_ref, b_ref, out):
    a = a_ref[0] + a_ref[1]
    b = b_ref[0] + b_ref[1]
    cnt = jnp.clip(a[:, 8:9], 1.0, None)
    mean = a[:, 0:8] / cnt
    msq = b[:, 0:8] / cnt
    var = jnp.clip(msq - mean * mean, 0.0, None)
    invstd = 1.0 / jnp.clip(jnp.sqrt(var), 1e-5, None)
    m16 = jnp.concatenate([mean, invstd], axis=1)
    place = (lax.broadcasted_iota(_i32, (16, D), 1)
             == lax.broadcasted_iota(_i32, (16, D), 0)).astype(_f32)
    out[...] = jnp.dot(m16, place, preferred_element_type=_f32)


def _statsn_tc(a2, b2):
    Rn = 2048
    return pl.pallas_call(
        _statsn_body,
        grid=(NPAD // Rn,),
        in_specs=[pl.BlockSpec((2, Rn, 16), lambda i: (0, i, 0)),
                  pl.BlockSpec((2, Rn, 16), lambda i: (0, i, 0))],
        out_specs=pl.BlockSpec((Rn, D), lambda i: (i, 0)),
        out_shape=jax.ShapeDtypeStruct((NPAD, D), _f32),
    )(a2, b2)


def _sc_sgather_body(sn_hbm, dst1d, sg_o, didx, b0, b1, g0, g1, w0, w1):
    c = lax.axis_index("c")
    s = lax.axis_index("s")
    wid = s * NC + c
    ngr = E // 128
    bufs = (b0, b1)
    gsems = (g0, g1)
    wsems = (w0, w1)

    def group(j, _):
        g = wid + NW * j

        @pl.when(g < ngr)
        def _():
            pltpu.sync_copy(dst1d.at[pl.ds(g * 128, 128)], didx)
            gds = [None, None]
            wds = [None, None]
            prev = None
            for t in range(8):
                pa = t % 2
                di = didx.at[pl.ds(t * 16, 16)]
                base = g * 128 + t * 16
                if wds[pa] is not None:
                    wds[pa].wait()
                gds[pa] = pltpu.async_copy(sn_hbm.at[di], bufs[pa], gsems[pa])
                if prev is not None:
                    gds[1 - pa].wait()
                    wds[1 - pa] = pltpu.async_copy(
                        bufs[1 - pa], sg_o.at[pl.ds(prev, 16)], wsems[1 - pa])
                prev = base
            gds[7 % 2].wait()
            wds[7 % 2] = pltpu.async_copy(
                bufs[7 % 2], sg_o.at[pl.ds(prev, 16)], wsems[7 % 2])
            wds[0].wait()
            wds[1].wait()
        return 0

    lax.fori_loop(0, (ngr + NW - 1) // NW, group, 0)


def _sc_sgather(sn, dst1d):
    return pl.kernel(
        _sc_sgather_body,
        out_type=jax.ShapeDtypeStruct((E, D), _f32),
        mesh=_mesh(),
        scratch_types=[
            pltpu.VMEM((128,), _i32),
            pltpu.VMEM((16, D), _f32),
            pltpu.VMEM((16, D), _f32),
            pltpu.SemaphoreType.DMA,
            pltpu.SemaphoreType.DMA,
            pltpu.SemaphoreType.DMA,
            pltpu.SemaphoreType.DMA,
        ],
    )(sn, dst1d)


def _attnw_body(s8, sg, vs, ev, ga, bi, m2, out):
    s = s8[...]
    mean = sg[:, 0:8]
    invstd = sg[:, 8:16]
    attn = ga[...] * (s - mean) * invstd + bi[...]
    rep = (lax.broadcasted_iota(_i32, (H, HD), 1) // QK
           == lax.broadcasted_iota(_i32, (H, HD), 0)).astype(_f32)
    w = jnp.dot(attn, rep, preferred_element_type=_f32) * (vs[...] + ev[...])
    mask = (lax.broadcasted_iota(_i32, (w.shape[0], 128), 1) // 64 == m2[...])
    for q in range(4):
        sl = w[:, q * 64:(q + 1) * 64]
        t2 = jnp.concatenate([sl, sl], axis=1)
        out[q] = jnp.where(mask, t2, 0.0)


def _attnw_tc(s8, sg, vs, ev, ga, bi, m2):
    R = 640
    return pl.pallas_call(
        _attnw_body,
        grid=(E // R,),
        in_specs=[pl.BlockSpec((R, H), lambda i: (i, 0)),
                  pl.BlockSpec((R, D), lambda i: (i, 0)),
                  pl.BlockSpec((R, HD), lambda i: (i, 0)),
                  pl.BlockSpec((R, HD), lambda i: (i, 0)),
                  pl.BlockSpec((1, H), lambda i: (0, 0)),
                  pl.BlockSpec((1, H), lambda i: (0, 0)),
                  pl.BlockSpec((R, 1), lambda i: (i, 0))],
        out_specs=pl.BlockSpec((4, R, 128), lambda i: (0, i, 0)),
        out_shape=jax.ShapeDtypeStruct((4, E, 128), _f32),
    )(s8, sg, vs, ev, ga, bi, m2)


def _sc_agg_body(w4_hbm, dq2, out_hbm, acc, zbuf, rowsb, idxb):
    c = lax.axis_index("c")
    s = lax.axis_index("s")
    nps = NP2 // NS

    def zrow(i, _):
        for jj in range(8):
            zbuf[i, jj * 16:(jj + 1) * 16] = jnp.zeros((16,), _f32)
        return 0

    lax.fori_loop(0, 8, zrow, 0)
    nch = E // 64

    for p in range(2):
        q = c + 2 * p

        def zcp(kk, _):
            pltpu.sync_copy(zbuf, acc.at[pl.ds(s * nps + kk * 8, 8)])
            return 0

        lax.fori_loop(0, nps // 8, zcp, 0)
        plsc.subcore_barrier()

        def chunk(j, _):
            r = s + NS * j

            @pl.when(r < nch)
            def _():
                pltpu.sync_copy(dq2.at[pl.ds(r * 64, 64)], idxb)
                pltpu.sync_copy(w4_hbm.at[q, pl.ds(r * 64, 64)], rowsb)
                pltpu.sync_copy(rowsb, acc.at[idxb], add=True)
            return 0

        lax.fori_loop(0, (nch + NS - 1) // NS, chunk, 0)
        plsc.subcore_barrier()
        pltpu.sync_copy(acc.at[pl.ds(s * nps, nps)],
                        out_hbm.at[q, pl.ds(s * nps, nps)])
        plsc.subcore_barrier()


def _sc_agg(w4, dq2):
    return pl.kernel(
        _sc_agg_body,
        out_type=jax.ShapeDtypeStruct((4, NP2, 128), _f32),
        mesh=_mesh(),
        scratch_types=[
            pltpu.VMEM_SHARED((NP2, 128), _f32),
            pltpu.VMEM((8, 128), _f32),
            pltpu.VMEM((64, 128), _f32),
            pltpu.VMEM((64,), _i32),
        ],
    )(w4, dq2)


def _aggstat_body(agg_ref, out):
    i = pl.program_id(0)
    a = agg_ref[...]
    s1 = jnp.sum(a, axis=0, keepdims=True)
    s2 = jnp.sum(a * a, axis=0, keepdims=True)
    upd = jnp.concatenate([s1, s2], axis=0)

    @pl.when(i == 0)
    def _():
        out[...] = jnp.zeros_like(out)

    out[...] += upd


def _aggstat_tc(agg2):
    Rn = 1000
    return pl.pallas_call(
        _aggstat_body,
        grid=(N // Rn,),
        in_specs=[pl.BlockSpec((Rn, HD), lambda i: (i, 0))],
        out_specs=pl.BlockSpec((2, HD), lambda i: (0, 0)),
        out_shape=jax.ShapeDtypeStruct((2, HD), _f32),
    )(agg2)


def _node_body(n_ref, agg_ref, stat, gnw, gnb, mixw, mixb,
               p1w, p1b, o1w, o1b, p2w, p2b, o2w, o2b, rz, out):
    agg = agg_ref[...]
    mean = stat[0:1] * np.float32(1.0 / N)
    msq = stat[1:2] * np.float32(1.0 / N)
    var = jnp.clip(msq - mean * mean, 0.0, None)
    std = jnp.sqrt(var + 1e-6)
    gn = gnw[...] * (agg - mean) / std + gnb[...]
    mixed = jnp.dot(jax.nn.relu(gn), mixw[...],
                    preferred_element_type=_f32) + mixb[...]
    r = rz[0, 0]
    n1 = n_ref[...] + r * mixed
    h1 = jnp.dot(n1, p1w[...], preferred_element_type=_f32) + p1b[...]
    a1 = h1[:, :HD] * jax.nn.relu(h1[:, HD:])
    f1 = jnp.dot(a1, o1w[...], preferred_element_type=_f32) + o1b[...]
    n2 = n1 + r * f1
    h2 = jnp.dot(n2, p2w[...], preferred_element_type=_f32) + p2b[...]
    a2 = h2[:, :HD] * jax.nn.relu(h2[:, HD:])
    f2 = jnp.dot(a2, o2w[...], preferred_element_type=_f32) + o2b[...]
    out[...] = n2 + r * f2


def _node_tc(n, agg2, stat, gnw, gnb, mixw, mixb,
             p1w, p1b, o1w, o1b, p2w, p2b, o2w, o2b, rz):
    Rn = 1000

    def const(shape):
        return pl.BlockSpec(shape, lambda i: tuple(0 for _ in shape))

    return pl.pallas_call(
        _node_body,
        grid=(N // Rn,),
        in_specs=[pl.BlockSpec((Rn, D), lambda i: (i, 0)),
                  pl.BlockSpec((Rn, HD), lambda i: (i, 0)),
                  const((2, HD)),
                  const((1, HD)), const((1, HD)),
                  const((HD, D)), const((1, D)),
                  const((D, 4 * D)), const((1, 4 * D)),
                  const((HD, D)), const((1, D)),
                  const((D, 4 * D)), const((1, 4 * D)),
                  const((HD, D)), const((1, D)),
                  const((1, 1))],
        out_specs=pl.BlockSpec((Rn, D), lambda i: (i, 0)),
        out_shape=jax.ShapeDtypeStruct((N, D), _f32),
    )(n, agg2, stat, gnw, gnb, mixw, mixb,
      p1w, p1b, o1w, o1b, p2w, p2b, o2w, o2b, rz)


def kernel(n, e, edge_index, q_w, q_b, k_w, k_b, v_w, v_b,
           eq_w, eq_b, ek_w, ek_b, ev_w, ev_b, gain, bias,
           node_ff_proj_w, node_ff_proj_b, node_ff_out_w, node_ff_out_b,
           edge_ff_proj_w, edge_ff_proj_b, edge_ff_out_w, edge_ff_out_b,
           node_ff2_proj_w, node_ff2_proj_b, node_ff2_out_w, node_ff2_out_b,
           edge_ff2_proj_w, edge_ff2_proj_b, edge_ff2_out_w, edge_ff2_out_b,
           gnw, gnb, mix_w, mix_b, rz_node, rz_edge):
    src1d = edge_index[0]
    dst1d = edge_index[1]
    m8 = (dst1d % 8).reshape(E, 1)
    m2 = (dst1d % 2).reshape(E, 1)
    dq8 = dst1d // 8
    dq2 = dst1d // 2
    row2 = lambda b: b.reshape(1, -1)
    rzn = rz_node.reshape(1, 1)
    rze = rz_edge.reshape(1, 1)

    eq, ek, ev, e2 = _edge_tc(
        e, eq_w.T, row2(eq_b), ek_w.T, row2(ek_b), ev_w.T, row2(ev_b),
        edge_ff_proj_w.T, row2(edge_ff_proj_b),
        edge_ff_out_w.T, row2(edge_ff_out_b),
        edge_ff2_proj_w.T, row2(edge_ff2_proj_b),
        edge_ff2_out_w.T, row2(edge_ff2_out_b), rze)

    q, k, v = _nodeproj_tc(n, q_w.T, row2(q_b), k_w.T, row2(k_b),
                           v_w.T, row2(v_b))

    qd, ks, vs = _sc_gather(q, k, v, src1d, dst1d)

    rA, rB, s8 = _score_tc(qd, eq, ks, ek, m8)

    outA, outB = _sc_stats(rA, rB, dq8)
    a2 = outA.reshape(NC, NPAD, 16)
    b2 = outB.reshape(NC, NPAD, 16)

    sn = _statsn_tc(a2, b2)

    sg = _sc_sgather(sn, dst1d)

    w4 = _attnw_tc(s8, sg, vs, ev, gain.reshape(1, H), bias.reshape(1, H), m2)

    out4 = _sc_agg(w4, dq2)
    agg2 = (out4.reshape(4, NP2, 2, 64)
            .transpose(1, 2, 0, 3).reshape(NPAD, HD))

    stat = _aggstat_tc(agg2)

    n3 = _node_tc(n, agg2, stat, row2(gnw), row2(gnb), mix_w.T, row2(mix_b),
                  node_ff_proj_w.T, row2(node_ff_proj_b),
                  node_ff_out_w.T, row2(node_ff_out_b),
                  node_ff2_proj_w.T, row2(node_ff2_proj_b),
                  node_ff2_out_w.T, row2(node_ff2_out_b), rzn)

    return n3, e2
